# Initial kernel scaffold; baseline (speedup 1.0000x reference)
import jax
import jax.numpy as jnp
from jax.experimental import pallas as pl


def kernel(x, conv1_w, bn1_g, bn1_b, conv2_w, bn2_g, bn2_b, conv3_w, bn3_g, bn3_b, conv4_w, bn4_g, bn4_b, conv5_w, bn5_g, bn5_b, lin1_w, bn6_g, bn6_b, lin2_w, lin2_b, bn7_g, bn7_b, lin3_w, lin3_b):
    raise NotImplementedError("write your pallas kernel here")


# speedup vs baseline: 2.0711x; 2.0711x over previous
import functools

import jax
import jax.numpy as jnp
from jax import lax
from jax.experimental import pallas as pl

_EPS = 1e-5
_K = 20
_SLOPE = 0.2


def _lrelu(t):
    return jnp.where(t >= 0, t, _SLOPE * t)


def _stats(p1, p2, cnt):
    mean = jnp.sum(p1, axis=0) / cnt
    ex2 = jnp.sum(p2, axis=0) / cnt
    var = ex2 - mean * mean
    return mean, var


def _apply_stage(mx, mn, p1, p2, g, b, cnt):
    mean, var = _stats(p1, p2, cnt)
    sel = jnp.where(g >= 0, mx, mn)
    return _lrelu((sel - mean) / jnp.sqrt(var + _EPS) * g + b)


def _edge_core(f, w, mx_o, mn_o, p1_o, p2_o):
    n = f.shape[1]
    o = w.shape[0]
    g = lax.dot_general(f, f, (((0,), (0,)), ((), ())),
                        preferred_element_type=jnp.float32)
    c = f.shape[0]
    if c <= 8:
        xx = f[0:1, :] * f[0:1, :]
        for ci in range(1, c):
            xx = xx + f[ci:ci + 1, :] * f[ci:ci + 1, :]
    else:
        xx = jnp.sum(f * f, axis=0, keepdims=True)
    inner = -2.0 * g
    pd = -xx - inner - xx.T
    iota_c = lax.broadcasted_iota(jnp.int32, (n, n), 1)

    def body(_, carry):
        pd, s1, s2, mx, mn = carry
        m = jnp.max(pd, axis=1, keepdims=True)
        cand = jnp.where(pd == m, iota_c, jnp.int32(n))
        pick = jnp.min(cand, axis=1, keepdims=True)
        onehot = iota_c == pick
        pd = jnp.where(onehot, -jnp.inf, pd)
        oh = onehot.astype(jnp.float32)
        feat = lax.dot_general(f, oh, (((1,), (1,)), ((), ())),
                               preferred_element_type=jnp.float32,
                               precision=lax.Precision.HIGHEST)
        cat = jnp.concatenate([feat - f, f], axis=0)
        t = jnp.dot(w, cat, preferred_element_type=jnp.float32)
        s1 = s1 + t
        s2 = s2 + t * t
        mx = jnp.maximum(mx, t)
        mn = jnp.minimum(mn, t)
        return pd, s1, s2, mx, mn

    init = (pd,
            jnp.zeros((o, n), jnp.float32),
            jnp.zeros((o, n), jnp.float32),
            jnp.full((o, n), -jnp.inf, jnp.float32),
            jnp.full((o, n), jnp.inf, jnp.float32))
    _, s1, s2, mx, mn = lax.fori_loop(0, _K, body, init)

    mx_o[0] = mx
    mn_o[0] = mn
    p1_o[0] = jnp.sum(s1, axis=1, keepdims=True)
    p2_o[0] = jnp.sum(s2, axis=1, keepdims=True)


def _edge_first_kernel(x_ref, w_ref, mx_o, mn_o, p1_o, p2_o):
    _edge_core(x_ref[0], w_ref[...], mx_o, mn_o, p1_o, p2_o)


def _edge_mid_kernel(mxp, mnp, p1p, p2p, gp, bp, w_ref,
                     mx_o, mn_o, p1_o, p2_o, *, cnt):
    f = _apply_stage(mxp[0], mnp[0], p1p[...], p2p[...], gp[...], bp[...], cnt)
    _edge_core(f, w_ref[...], mx_o, mn_o, p1_o, p2_o)


def _edge_outs(bsz, o, n):
    shapes = [jax.ShapeDtypeStruct((bsz, o, n), jnp.float32)] * 2 + \
             [jax.ShapeDtypeStruct((bsz, o, 1), jnp.float32)] * 2
    specs = [pl.BlockSpec((1, o, n), lambda b: (b, 0, 0))] * 2 + \
            [pl.BlockSpec((1, o, 1), lambda b: (b, 0, 0))] * 2
    return shapes, specs


def _edge_first(x, w):
    bsz, c, n = x.shape
    o = w.shape[0]
    out_shape, out_specs = _edge_outs(bsz, o, n)
    return pl.pallas_call(
        _edge_first_kernel,
        grid=(bsz,),
        in_specs=[
            pl.BlockSpec((1, c, n), lambda b: (b, 0, 0)),
            pl.BlockSpec((o, 2 * c), lambda b: (0, 0)),
        ],
        out_specs=out_specs,
        out_shape=out_shape,
    )(x, w)


def _edge_mid(prev, g, b, w, cnt):
    mxp, mnp, p1p, p2p = prev
    bsz, cp, n = mxp.shape
    o = w.shape[0]
    out_shape, out_specs = _edge_outs(bsz, o, n)
    return pl.pallas_call(
        functools.partial(_edge_mid_kernel, cnt=cnt),
        grid=(bsz,),
        in_specs=[
            pl.BlockSpec((1, cp, n), lambda b: (b, 0, 0)),
            pl.BlockSpec((1, cp, n), lambda b: (b, 0, 0)),
            pl.BlockSpec((bsz, cp, 1), lambda b: (0, 0, 0)),
            pl.BlockSpec((bsz, cp, 1), lambda b: (0, 0, 0)),
            pl.BlockSpec((cp, 1), lambda b: (0, 0)),
            pl.BlockSpec((cp, 1), lambda b: (0, 0)),
            pl.BlockSpec((o, 2 * cp), lambda b: (0, 0)),
        ],
        out_specs=out_specs,
        out_shape=out_shape,
    )(mxp, mnp, p1p, p2p, g.reshape(cp, 1), b.reshape(cp, 1), w)


def _head1_kernel(*refs, cnt):
    ins, (f5_o, p1_o, p2_o) = refs[:-3], refs[-3:]
    w5 = ins[24]
    feats = []
    for i in range(4):
        mxp, mnp, p1p, p2p, gp, bp = ins[i * 6:(i + 1) * 6]
        feats.append(_apply_stage(mxp[0], mnp[0], p1p[...], p2p[...],
                                  gp[...], bp[...], cnt))
    x5 = jnp.concatenate(feats, axis=0)
    acc = jnp.dot(w5[...], x5, preferred_element_type=jnp.float32)
    f5_o[0] = acc
    p1_o[0] = jnp.sum(acc, axis=1, keepdims=True)
    p2_o[0] = jnp.sum(acc * acc, axis=1, keepdims=True)


def _head1(stages, bns, w5, cnt):
    bsz, _, n = stages[0][0].shape
    o5 = w5.shape[0]
    in_specs = []
    args = []
    for (mxp, mnp, p1p, p2p), (g, b) in zip(stages, bns):
        cp = mxp.shape[1]
        in_specs += [
            pl.BlockSpec((1, cp, n), lambda b: (b, 0, 0)),
            pl.BlockSpec((1, cp, n), lambda b: (b, 0, 0)),
            pl.BlockSpec((bsz, cp, 1), lambda b: (0, 0, 0)),
            pl.BlockSpec((bsz, cp, 1), lambda b: (0, 0, 0)),
            pl.BlockSpec((cp, 1), lambda b: (0, 0)),
            pl.BlockSpec((cp, 1), lambda b: (0, 0)),
        ]
        args += [mxp, mnp, p1p, p2p, g.reshape(cp, 1), b.reshape(cp, 1)]
    in_specs.append(pl.BlockSpec(w5.shape, lambda b: (0, 0)))
    args.append(w5)
    out_shape = [jax.ShapeDtypeStruct((bsz, o5, n), jnp.float32),
                 jax.ShapeDtypeStruct((bsz, o5, 1), jnp.float32),
                 jax.ShapeDtypeStruct((bsz, o5, 1), jnp.float32)]
    out_specs = [pl.BlockSpec((1, o5, n), lambda b: (b, 0, 0)),
                 pl.BlockSpec((1, o5, 1), lambda b: (b, 0, 0)),
                 pl.BlockSpec((1, o5, 1), lambda b: (b, 0, 0))]
    return pl.pallas_call(
        functools.partial(_head1_kernel, cnt=cnt),
        grid=(bsz,),
        in_specs=in_specs,
        out_specs=out_specs,
        out_shape=out_shape,
    )(*args)


def _pool_kernel(f5_ref, p1_ref, p2_ref, g5, b5, f6_o, f7_o, *, cnt5):
    mean5, var5 = _stats(p1_ref[...], p2_ref[...], cnt5)
    n = f5_ref.shape[2]
    a = _lrelu((f5_ref[0] - mean5) / jnp.sqrt(var5 + _EPS) * g5[...] + b5[...])
    f6_o[0] = jnp.max(a, axis=1, keepdims=True).T
    f7_o[0] = (jnp.sum(a, axis=1, keepdims=True) / n).T


def _head2_kernel(f6_ref, f7_ref, w1a, w1b, g6, b6,
                  w2t, l2b, g7, b7, w3t, l3b, out_ref):
    f6 = f6_ref[...]
    f7 = f7_ref[...]
    h = jnp.dot(jnp.concatenate([f6, f7], axis=1),
                jnp.concatenate([w1a[...], w1b[...]], axis=0),
                preferred_element_type=jnp.float32)
    m = jnp.mean(h, axis=0, keepdims=True)
    v = jnp.mean((h - m) ** 2, axis=0, keepdims=True)
    h = _lrelu((h - m) / jnp.sqrt(v + _EPS) * g6[...] + b6[...])
    h2 = jnp.dot(h, w2t[...], preferred_element_type=jnp.float32) + l2b[...]
    m2 = jnp.mean(h2, axis=0, keepdims=True)
    v2 = jnp.mean((h2 - m2) ** 2, axis=0, keepdims=True)
    h2 = _lrelu((h2 - m2) / jnp.sqrt(v2 + _EPS) * g7[...] + b7[...])
    out_ref[...] = (jnp.dot(h2, w3t[...], preferred_element_type=jnp.float32)
                    + l3b[...])


def _head2(f5, p1, p2, bn5_g, bn5_b, lin1_w, bn6_g, bn6_b,
           lin2_w, lin2_b, bn7_g, bn7_b, lin3_w, lin3_b):
    bsz, o5, n = f5.shape
    d1 = lin1_w.shape[0]
    d2 = lin2_w.shape[0]
    d3 = lin3_w.shape[0]
    f6, f7 = pl.pallas_call(
        functools.partial(_pool_kernel, cnt5=float(bsz * n)),
        grid=(bsz,),
        in_specs=[
            pl.BlockSpec((1, o5, n), lambda b: (b, 0, 0)),
            pl.BlockSpec((bsz, o5, 1), lambda b: (0, 0, 0)),
            pl.BlockSpec((bsz, o5, 1), lambda b: (0, 0, 0)),
            pl.BlockSpec((o5, 1), lambda b: (0, 0)),
            pl.BlockSpec((o5, 1), lambda b: (0, 0)),
        ],
        out_specs=[pl.BlockSpec((1, 1, o5), lambda b: (b, 0, 0)),
                   pl.BlockSpec((1, 1, o5), lambda b: (b, 0, 0))],
        out_shape=[jax.ShapeDtypeStruct((bsz, 1, o5), jnp.float32),
                   jax.ShapeDtypeStruct((bsz, 1, o5), jnp.float32)],
    )(f5, p1, p2, bn5_g.reshape(o5, 1), bn5_b.reshape(o5, 1))
    f6 = f6.reshape(bsz, o5)
    f7 = f7.reshape(bsz, o5)

    w1a = lin1_w[:, :o5].T
    w1b = lin1_w[:, o5:].T
    args = [f6, f7, w1a, w1b, bn6_g.reshape(1, d1), bn6_b.reshape(1, d1),
            lin2_w.T, lin2_b.reshape(1, d2),
            bn7_g.reshape(1, d2), bn7_b.reshape(1, d2),
            lin3_w.T, lin3_b.reshape(1, d3)]
    in_specs = [pl.BlockSpec(a.shape, functools.partial(lambda nd: (0,) * nd, a.ndim))
                for a in args]
    return pl.pallas_call(
        _head2_kernel,
        grid=(),
        in_specs=in_specs,
        out_specs=pl.BlockSpec((bsz, d3), lambda: (0, 0)),
        out_shape=jax.ShapeDtypeStruct((bsz, d3), jnp.float32),
    )(*args)


def kernel(x, conv1_w, bn1_g, bn1_b, conv2_w, bn2_g, bn2_b, conv3_w, bn3_g,
           bn3_b, conv4_w, bn4_g, bn4_b, conv5_w, bn5_g, bn5_b, lin1_w,
           bn6_g, bn6_b, lin2_w, lin2_b, bn7_g, bn7_b, lin3_w, lin3_b):
    bsz, _, n = x.shape
    cnt = float(bsz * n * _K)

    r1 = _edge_first(x, conv1_w)
    r2 = _edge_mid(r1, bn1_g, bn1_b, conv2_w, cnt=cnt)
    r3 = _edge_mid(r2, bn2_g, bn2_b, conv3_w, cnt=cnt)
    r4 = _edge_mid(r3, bn3_g, bn3_b, conv4_w, cnt=cnt)
    f5, p1, p2 = _head1(
        [r1, r2, r3, r4],
        [(bn1_g, bn1_b), (bn2_g, bn2_b), (bn3_g, bn3_b), (bn4_g, bn4_b)],
        conv5_w, cnt)
    return _head2(f5, p1, p2, bn5_g, bn5_b, lin1_w, bn6_g, bn6_b,
                  lin2_w, lin2_b, bn7_g, bn7_b, lin3_w, lin3_b)
